# Initial kernel scaffold; baseline (speedup 1.0000x reference)
import jax
import jax.numpy as jnp
from jax.experimental import pallas as pl


def kernel(x, edge_index, batch, params):
    raise NotImplementedError("write your pallas kernel here")


# speedup vs baseline: 6.4115x; 6.4115x over previous
import functools

import jax
import jax.numpy as jnp
from jax import lax
from jax.experimental import pallas as pl
from jax.experimental.pallas import tpu as pltpu
from jax.experimental.pallas import tpu_sc as plsc

_N, _E, _D = 10000, 320000, 128
_ER = _E // 128
_NW = 32
_NP = 10112
_R = _NP // 128
_K1, _K2 = 5000, 2500
_F32 = jnp.float32
_I32 = jnp.int32


def _mesh():
    return plsc.VectorSubcoreMesh(core_axis_name="c", subcore_axis_name="s")


def _agg_sc():
    rps = _NP // 16

    @functools.partial(
        pl.kernel, mesh=_mesh(),
        out_type=jax.ShapeDtypeStruct((2, _NP, 128), _F32),
        scratch_types=[
            pltpu.VMEM((128,), _I32),
            pltpu.VMEM((128,), _I32),
            pltpu.VMEM((128, 128), _F32),
            pltpu.VMEM_SHARED((_NP, 128), _F32),
            pltpu.SemaphoreType.DMA,
        ],
    )
    def k(x_hbm, src_hbm, dst_hbm, z_hbm, out_hbm, sidx, didx, rows, acc, sem):
        c = lax.axis_index("c")
        s = lax.axis_index("s")
        w = s * 2 + c
        pltpu.sync_copy(z_hbm.at[pl.ds(s * rps, rps)], acc.at[pl.ds(s * rps, rps)])
        plsc.subcore_barrier()

        def body(j, carry):
            pltpu.sync_copy(src_hbm.at[j], sidx)
            pltpu.sync_copy(dst_hbm.at[j], didx)
            pltpu.async_copy(x_hbm.at[sidx], rows, sem).wait()
            pltpu.sync_copy(rows, acc.at[didx], add=True)
            return carry

        lax.fori_loop(w * _ER // _NW, (w + 1) * _ER // _NW, body, 0)
        plsc.subcore_barrier()
        pltpu.sync_copy(acc.at[pl.ds(s * rps, rps)],
                        out_hbm.at[c, pl.ds(s * rps, rps)])

    return k


def _eye128():
    a = lax.broadcasted_iota(_I32, (128, 128), 0)
    b = lax.broadcasted_iota(_I32, (128, 128), 1)
    return (a == b).astype(_F32)


def _col(row):
    return lax.dot_general(_eye128(), row, (((1,), (1,)), ((), ())),
                           preferred_element_type=_F32)


def _row(col):
    return lax.dot_general(col, _eye128(), (((0,), (0,)), ((), ())),
                           preferred_element_type=_F32)


def _dense_tc():
    def body(agg_ref, degp_ref, x_ref, sel_ref, wl_ref, wr_ref, b_ref, o_ref):
        a = agg_ref[0] + agg_ref[1]
        deg = jnp.maximum(degp_ref[0] + degp_ref[1], 1.0)
        mean = a / deg
        out = (jnp.dot(mean, wl_ref[...], preferred_element_type=_F32)
               + jnp.dot(x_ref[...], wr_ref[...], preferred_element_type=_F32)
               + b_ref[...])
        nrm = jnp.sqrt(jnp.sum(out * out, axis=1, keepdims=True))
        out = out / jnp.maximum(nrm, 1e-12)
        out = jnp.maximum(out, 0.0)
        o_ref[...] = out * sel_ref[...]

    return pl.pallas_call(
        body,
        grid=(_R,),
        in_specs=[
            pl.BlockSpec((2, 128, 128), lambda i: (0, i, 0)),
            pl.BlockSpec((2, 128, 128), lambda i: (0, i, 0)),
            pl.BlockSpec((128, 128), lambda i: (i, 0)),
            pl.BlockSpec((128, 128), lambda i: (i, 0)),
            pl.BlockSpec((128, 128), lambda i: (0, 0)),
            pl.BlockSpec((128, 128), lambda i: (0, 0)),
            pl.BlockSpec((1, 128), lambda i: (0, 0)),
        ],
        out_specs=pl.BlockSpec((128, 128), lambda i: (i, 0)),
        out_shape=jax.ShapeDtypeStruct((_NP, 128), _F32),
    )


def _scoreprep_tc():
    def body(aggp_ref, degp_ref, h_ref, wl_ref, wr_ref, b_ref, s_ref):
        aggsum = aggp_ref[0] + aggp_ref[1]
        deg = jnp.maximum((degp_ref[0] + degp_ref[1])[:, 0:1], 1.0)
        zcol = jnp.dot(aggsum, wl_ref[...], preferred_element_type=_F32)
        rcol = jnp.dot(h_ref[...], wr_ref[...], preferred_element_type=_F32)
        scol = jnp.tanh(zcol / deg + rcol + b_ref[0, 0])
        s_ref[...] = _row(scol).reshape(1, 1, 128)

    return pl.pallas_call(
        body,
        grid=(_R,),
        in_specs=[
            pl.BlockSpec((2, 128, 128), lambda i: (0, i, 0)),
            pl.BlockSpec((2, 128, 128), lambda i: (0, i, 0)),
            pl.BlockSpec((128, 128), lambda i: (i, 0)),
            pl.BlockSpec((128, 1), lambda i: (0, 0)),
            pl.BlockSpec((128, 1), lambda i: (0, 0)),
            pl.BlockSpec((1, 1), lambda i: (0, 0)),
        ],
        out_specs=pl.BlockSpec((1, 1, 128), lambda i: (i, 0, 0)),
        out_shape=jax.ShapeDtypeStruct((_R, 1, 128), _F32),
    )


def _score_tc(kk):
    def body(s_ref, selp_ref, m_ref):
        s = s_ref[...].reshape(_R, 128)
        rid = (lax.broadcasted_iota(_I32, (_R, 128), 0) * 128
               + lax.broadcasted_iota(_I32, (_R, 128), 1))
        valid = selp_ref[...].reshape(_R, 128) > 0.5
        sm = jnp.where(valid, s, -jnp.inf)
        bits = lax.bitcast_convert_type(sm, jnp.uint32)
        key = jnp.where((bits >> 31) == jnp.uint32(1), ~bits,
                        bits | jnp.uint32(0x80000000))

        def bis(j, t):
            cand = t | (jnp.uint32(1) << (jnp.uint32(31) - j.astype(jnp.uint32)))
            cnt = jnp.sum((key >= cand).astype(_I32))
            return jnp.where(cnt >= kk, cand, t)

        t = lax.fori_loop(0, 32, bis, jnp.uint32(0))
        gt = key > t
        tie = key == t
        need = kk - jnp.sum(gt.astype(_I32))

        def bis2(j, c):
            cand = c | (1 << (13 - j))
            cntt = jnp.sum((tie & (rid < cand)).astype(_I32))
            return jnp.where(cntt < need, cand, c)

        c = lax.fori_loop(0, 14, bis2, jnp.int32(0))
        mask = gt | (tie & (rid <= c) & (need > 0))
        m_ref[...] = mask.astype(_F32).reshape(_R, 1, 128)

    return pl.pallas_call(
        body,
        in_specs=[
            pl.BlockSpec((_R, 1, 128), lambda: (0, 0, 0)),
            pl.BlockSpec((_R, 1, 128), lambda: (0, 0, 0)),
        ],
        out_specs=pl.BlockSpec((_R, 1, 128), lambda: (0, 0, 0)),
        out_shape=jax.ShapeDtypeStruct((_R, 1, 128), _F32),
    )


def _hs_tc():
    def body(h_ref, s_ref, sel_ref, o_ref, sel2d_ref):
        scol = _col(s_ref[...][0])
        selcol = _col(sel_ref[...][0])
        o_ref[...] = h_ref[...] * scol * selcol
        sel2d_ref[...] = jnp.broadcast_to(selcol, (128, 128))

    return pl.pallas_call(
        body,
        grid=(_R,),
        in_specs=[
            pl.BlockSpec((128, 128), lambda i: (i, 0)),
            pl.BlockSpec((1, 1, 128), lambda i: (i, 0, 0)),
            pl.BlockSpec((1, 1, 128), lambda i: (i, 0, 0)),
        ],
        out_specs=[
            pl.BlockSpec((128, 128), lambda i: (i, 0)),
            pl.BlockSpec((128, 128), lambda i: (i, 0)),
        ],
        out_shape=[
            jax.ShapeDtypeStruct((_NP, 128), _F32),
            jax.ShapeDtypeStruct((_NP, 128), _F32),
        ],
    )


def _rmax_tc():
    def body(h_ref, sel_ref, o_ref):
        o_ref[...] = jnp.max(jnp.where(sel_ref[...] > 0.5, h_ref[...],
                                       -jnp.inf), axis=0, keepdims=True)

    return pl.pallas_call(
        body,
        in_specs=[
            pl.BlockSpec((_NP, 128), lambda: (0, 0)),
            pl.BlockSpec((_NP, 128), lambda: (0, 0)),
        ],
        out_specs=pl.BlockSpec((1, 128), lambda: (0, 0)),
        out_shape=jax.ShapeDtypeStruct((1, 128), _F32),
    )


def _mlp_tc():
    def body(c_ref, w1_ref, b1_ref, w2_ref, b2_ref, o_ref):
        hp = jnp.maximum(
            jnp.dot(c_ref[...], w1_ref[...], preferred_element_type=_F32)
            + b1_ref[...], 0.0)
        o_ref[...] = (jnp.dot(hp, w2_ref[...], preferred_element_type=_F32)
                      + b2_ref[...])

    return pl.pallas_call(
        body,
        in_specs=[
            pl.BlockSpec((1, 384), lambda: (0, 0)),
            pl.BlockSpec((384, 50), lambda: (0, 0)),
            pl.BlockSpec((1, 50), lambda: (0, 0)),
            pl.BlockSpec((50, 10), lambda: (0, 0)),
            pl.BlockSpec((1, 10), lambda: (0, 0)),
        ],
        out_specs=pl.BlockSpec((1, 10), lambda: (0, 0)),
        out_shape=jax.ShapeDtypeStruct((1, 10), _F32),
    )


def kernel(x, edge_index, batch, params):
    p = params
    del batch
    zeros = jnp.zeros((_NP, 128), _F32)
    xp = jnp.pad(x, ((0, _NP - _N), (0, 0)))
    src = edge_index[0].reshape(_ER, 128)
    dst = edge_index[1].reshape(_ER, 128)

    sel = (lax.iota(_I32, _NP) < _N).astype(_F32)
    sel_rm = sel.reshape(_R, 1, 128)
    sel2d = jnp.broadcast_to(sel[:, None], (_NP, 128))

    agg = _agg_sc()
    degp = agg(sel2d, src, dst, zeros)

    h = xp
    outs = []
    for nm in ['11', '12', '13']:
        a = agg(h, src, dst, zeros)
        h = _dense_tc()(a, degp, h, sel2d, p['W' + nm + '_l'],
                        p['W' + nm + '_r'], p['b' + nm].reshape(1, 128))
    outs.append(_rmax_tc()(h, sel2d))

    for pool_nm, blk, kn in [('p1', ['21', '22', '23'], _K1),
                             ('p2', ['31', '32', '33'], _K2)]:
        sagg = agg(h, src, dst, zeros)
        s_rm = _scoreprep_tc()(sagg, degp, h, p['W' + pool_nm + '_l'],
                               p['W' + pool_nm + '_r'],
                               p['b' + pool_nm].reshape(1, 1))
        sel_rm = _score_tc(kn)(s_rm, sel_rm)
        h, sel2d = _hs_tc()(h, s_rm, sel_rm)
        degp = agg(sel2d, src, dst, zeros)
        for nm in blk:
            a = agg(h, src, dst, zeros)
            h = _dense_tc()(a, degp, h, sel2d, p['W' + nm + '_l'],
                            p['W' + nm + '_r'], p['b' + nm].reshape(1, 128))
        outs.append(_rmax_tc()(h, sel2d))

    cat = jnp.concatenate(outs, axis=1)
    return _mlp_tc()(cat, p['P1_W'], p['P1_b'].reshape(1, 50),
                     p['P2_W'], p['P2_b'].reshape(1, 10))
